# baseline (speedup 1.0000x reference)
import functools

import jax
import jax.numpy as jnp
from jax.experimental import pallas as pl
from jax.experimental.pallas import tpu as pltpu


def _conv_body(x_hbm, w_ref, b_ref, *refs, th, n_rt, w_out, cin, rg, fold,
               emit_preact, do_pool):
    if emit_preact:
        y_ref, f_ref, x_vmem, sem = refs
    else:
        y_ref, x_vmem, sem = refs
        f_ref = None

    n = pl.program_id(0)
    j = pl.program_id(1)
    rt = pl.program_id(2)
    dt = x_vmem.dtype

    if n_rt == 1:
        @pl.when(j == 0)
        def _():
            x_vmem[0:1] = jnp.zeros((1, w_out, cin), dt)
            x_vmem[th + 1:th + 2] = jnp.zeros((1, w_out, cin), dt)
            cp = pltpu.make_async_copy(x_hbm.at[n], x_vmem.at[pl.ds(1, th)],
                                       sem)
            cp.start()
            cp.wait()

        def row(rr):
            return x_vmem[rr]
    else:
        slot = jax.lax.rem(rt, 2)

        def halo(rt_t, s, mode):
            def go(cp):
                cp.start() if mode == 'start' else cp.wait()

            if isinstance(rt_t, int):
                assert rt_t == 0 and s == 0
                go(pltpu.make_async_copy(
                    x_hbm.at[n, pl.ds(0, th + 1)],
                    x_vmem.at[0, pl.ds(1, th + 1)], sem.at[0]))
                return
            first = rt_t == 0
            last = rt_t == n_rt - 1
            r0_t = rt_t * th

            @pl.when(first)
            def _():
                go(pltpu.make_async_copy(
                    x_hbm.at[n, pl.ds(0, th + 1)],
                    x_vmem.at[s, pl.ds(1, th + 1)], sem.at[s]))

            @pl.when(jnp.logical_and(jnp.logical_not(first),
                                     jnp.logical_not(last)))
            def _():
                go(pltpu.make_async_copy(
                    x_hbm.at[n, pl.ds(r0_t - 1, th + 2)], x_vmem.at[s],
                    sem.at[s]))

            @pl.when(jnp.logical_and(last, jnp.logical_not(first)))
            def _():
                go(pltpu.make_async_copy(
                    x_hbm.at[n, pl.ds(r0_t - 1, th + 1)],
                    x_vmem.at[s, pl.ds(0, th + 1)], sem.at[s]))

        @pl.when(rt == 0)
        def _():
            halo(0, 0, 'start')

        @pl.when(rt + 1 < n_rt)
        def _():
            halo(rt + 1, 1 - slot, 'start')

        halo(rt, slot, 'wait')
        @pl.when(rt == 0)
        def _():
            x_vmem[0, 0:1] = jnp.zeros((1, w_out, cin), dt)

        @pl.when(rt == n_rt - 1)
        def _():
            x_vmem[(n_rt - 1) % 2, th + 1:th + 2] = jnp.zeros(
                (1, w_out, cin), dt)

        def row(rr):
            return x_vmem[slot, rr]

    bias = b_ref[...]
    tco = b_ref.shape[-1]
    if fold:
        w_all = w_ref[...]
    else:
        w_taps = [w_ref[t] for t in range(9)]

    zcol = jnp.zeros((1, cin), dt)

    def prow(rr):
        return jnp.concatenate([zcol, row(rr), zcol], axis=0)

    for g in range(th // rg):
        r0 = g * rg
        pr = [prow(r0 + i) for i in range(rg + 2)]
        if fold:
            lhs = jnp.concatenate(
                [jnp.concatenate([pr[i + dy][dx:dx + w_out]
                                  for dy in range(3) for dx in range(3)],
                                 axis=-1)
                 for i in range(rg)], axis=0)
            acc = jnp.dot(lhs, w_all, preferred_element_type=jnp.float32)
        else:
            acc = jnp.zeros((rg * w_out, tco), jnp.float32)
            t = 0
            for dy in range(3):
                for dx in range(3):
                    if rg == 1:
                        l = pr[dy][dx:dx + w_out]
                    else:
                        l = jnp.concatenate(
                            [pr[i + dy][dx:dx + w_out] for i in range(rg)],
                            axis=0)
                    acc = acc + jnp.dot(l, w_taps[t],
                                        preferred_element_type=jnp.float32)
                    t += 1
        acc = acc + bias
        if emit_preact:
            f_ref[:, pl.ds(r0, rg)] = jnp.transpose(acc).reshape(
                tco, rg, w_out)
        y = jnp.maximum(acc, 0.0)
        if do_pool:
            y4 = y.reshape(rg // 2, 2, w_out, tco)
            m = jnp.maximum(y4[:, 0], y4[:, 1])
            m4 = m.reshape(rg // 2, w_out // 2, 2, tco)
            m = jnp.maximum(m4[:, :, 0, :], m4[:, :, 1, :])
            y_ref[pl.ds(r0 // 2, rg // 2)] = m.astype(y_ref.dtype)
        else:
            y_ref[pl.ds(r0, rg)] = y.reshape(rg, w_out, tco).astype(y_ref.dtype)


def _conv(x, w, b, *, preact, pool):
    n, h, wd, cin = x.shape
    cout = w.shape[-1]

    th = 8 if h % 8 == 0 else h
    n_rt = h // th
    tco = min(cout, 128)
    n_co = cout // tco
    fold = (cin % 128 == 0)

    rg = th
    for d in range(1, th + 1):
        if th % d == 0 and d * wd >= 112:
            rg = d
            break
    if pool and rg % 2:
        rg *= 2
    assert th % rg == 0 and (not pool or rg % 2 == 0)

    wb = w.astype(jnp.bfloat16)
    if fold:
        w_in = wb.reshape(9 * cin, cout)
        w_spec = pl.BlockSpec((9 * cin, tco), lambda i, j, k: (0, j))
    else:
        w_in = wb.reshape(9, cin, cout)
        w_spec = pl.BlockSpec((9, cin, tco), lambda i, j, k: (0, 0, j))
    b_in = b.reshape(1, cout)

    ho, wo = (h // 2, wd // 2) if pool else (h, wd)
    tho = th // 2 if pool else th
    y_sds = jax.ShapeDtypeStruct((n, ho, wo, cout), jnp.bfloat16)
    y_spec = pl.BlockSpec((None, tho, wo, tco), lambda i, j, k: (i, k, 0, j))
    if preact:
        f_sds = jax.ShapeDtypeStruct((n, cout, h, wd), jnp.float32)
        f_spec = pl.BlockSpec((None, tco, th, wd), lambda i, j, k: (i, j, k, 0))
        out_shape = (y_sds, f_sds)
        out_specs = (y_spec, f_spec)
    else:
        out_shape = y_sds
        out_specs = y_spec

    body = functools.partial(_conv_body, th=th, n_rt=n_rt, w_out=wd, cin=cin,
                             rg=rg, fold=fold, emit_preact=preact, do_pool=pool)
    outs = pl.pallas_call(
        body,
        out_shape=out_shape,
        grid_spec=pltpu.PrefetchScalarGridSpec(
            num_scalar_prefetch=0,
            grid=(n, n_co, n_rt),
            in_specs=[
                pl.BlockSpec(memory_space=pl.ANY),
                w_spec,
                pl.BlockSpec((1, tco), lambda i, j, k: (0, j)),
            ],
            out_specs=out_specs,
            scratch_shapes=[
                pltpu.VMEM((th + 2, wd, cin), jnp.bfloat16) if n_rt == 1
                else pltpu.VMEM((2, th + 2, wd, cin), jnp.bfloat16),
                pltpu.SemaphoreType.DMA if n_rt == 1
                else pltpu.SemaphoreType.DMA((2,)),
            ]),
        compiler_params=pltpu.CompilerParams(
            dimension_semantics=("parallel", "parallel", "arbitrary")),
    )(x, w_in, b_in)
    if preact:
        return outs[0], outs[1]
    return outs, None


_PLAN = [(True, False), (False, True),
         (True, False), (False, True),
         (True, False), (False, False), (False, False), (False, True),
         (True, False), (False, False), (False, False), (False, True),
         (True, False)]


def kernel(x, w0, b0, w1, b1, w2, b2, w3, b3, w4, b4, w5, b5, w6, b6, w7, b7,
           w8, b8, w9, b9, w10, b10, w11, b11, w12, b12, w13, b13, w14, b14,
           w15, b15):
    ws = [w0, w1, w2, w3, w4, w5, w6, w7, w8, w9, w10, w11, w12]
    bs = [b0, b1, b2, b3, b4, b5, b6, b7, b8, b9, b10, b11, b12]
    x = jnp.transpose(x, (0, 2, 3, 1)).astype(jnp.bfloat16)
    feats = []
    for li, (pre, po) in enumerate(_PLAN):
        x, f = _conv(x, ws[li], bs[li], preact=pre, pool=po)
        if pre:
            feats.append(f)
    return tuple(feats)


# speedup vs baseline: 1.0047x; 1.0047x over previous
import functools

import jax
import jax.numpy as jnp
from jax.experimental import pallas as pl
from jax.experimental.pallas import tpu as pltpu


def _conv_body(x_hbm, w_ref, b_ref, *refs, th, n_rt, w_out, cin, rg, fold,
               pad_scr, emit_preact, do_pool):
    if emit_preact:
        y_ref, f_ref, x_vmem, sem = refs
    else:
        y_ref, x_vmem, sem = refs
        f_ref = None

    n = pl.program_id(0)
    j = pl.program_id(1)
    rt = pl.program_id(2)
    dt = x_vmem.dtype
    wp = x_vmem.shape[-2]
    col0 = 8 if pad_scr else 0

    if n_rt == 1:
        @pl.when(j == 0)
        def _():
            x_vmem[0:1] = jnp.zeros((1, wp, cin), dt)
            x_vmem[th + 1:th + 2] = jnp.zeros((1, wp, cin), dt)
            cp = pltpu.make_async_copy(x_hbm.at[n], x_vmem.at[pl.ds(1, th)],
                                       sem)
            cp.start()
            cp.wait()

        def rows(a, m):
            return x_vmem[a:a + m]
    else:
        slot = jax.lax.rem(rt, 2)

        def dst(s, a, m):
            if pad_scr:
                return x_vmem.at[s, pl.ds(a, m), pl.ds(col0, w_out)]
            return x_vmem.at[s, pl.ds(a, m)]

        def halo(rt_t, s, mode):
            def go(cp):
                cp.start() if mode == 'start' else cp.wait()

            if isinstance(rt_t, int):
                assert rt_t == 0 and s == 0
                go(pltpu.make_async_copy(
                    x_hbm.at[n, pl.ds(0, th + 1)], dst(0, 1, th + 1),
                    sem.at[0]))
                return
            first = rt_t == 0
            last = rt_t == n_rt - 1
            r0_t = rt_t * th

            @pl.when(first)
            def _():
                go(pltpu.make_async_copy(
                    x_hbm.at[n, pl.ds(0, th + 1)], dst(s, 1, th + 1),
                    sem.at[s]))

            @pl.when(jnp.logical_and(jnp.logical_not(first),
                                     jnp.logical_not(last)))
            def _():
                go(pltpu.make_async_copy(
                    x_hbm.at[n, pl.ds(r0_t - 1, th + 2)], dst(s, 0, th + 2),
                    sem.at[s]))

            @pl.when(jnp.logical_and(last, jnp.logical_not(first)))
            def _():
                go(pltpu.make_async_copy(
                    x_hbm.at[n, pl.ds(r0_t - 1, th + 1)], dst(s, 0, th + 1),
                    sem.at[s]))

        @pl.when(rt == 0)
        def _():
            halo(0, 0, 'start')

        @pl.when(rt + 1 < n_rt)
        def _():
            halo(rt + 1, 1 - slot, 'start')

        halo(rt, slot, 'wait')

        @pl.when(rt == 0)
        def _():
            x_vmem[0, 0:1] = jnp.zeros((1, wp, cin), dt)

        @pl.when(rt == n_rt - 1)
        def _():
            x_vmem[(n_rt - 1) % 2, th + 1:th + 2] = jnp.zeros(
                (1, wp, cin), dt)

        if pad_scr:
            x_vmem[slot, :, col0 - 1:col0] = jnp.zeros((th + 2, 1, cin), dt)
            x_vmem[slot, :, col0 + w_out:col0 + w_out + 1] = jnp.zeros(
                (th + 2, 1, cin), dt)

        def rows(a, m):
            return x_vmem[slot, a:a + m]

    bias = b_ref[...]
    tco = b_ref.shape[-1]
    if fold:
        w_all = w_ref[...]
    else:
        w_taps = [w_ref[t] for t in range(9)]

    zcol = jnp.zeros((1, cin), dt)

    def prow(rr):
        return jnp.concatenate([zcol, rows(rr, 1).reshape(wp, cin), zcol],
                               axis=0)

    for g in range(th // rg):
        r0 = g * rg
        if pad_scr:
            acc = jnp.zeros((rg * w_out, tco), jnp.float32)
            for t, (dy, dx) in enumerate([(a, b) for a in range(3)
                                          for b in range(3)]):
                lhs = rows(r0 + dy, rg)[:, col0 - 1 + dx:col0 - 1 + dx + w_out]
                acc = acc + jnp.dot(lhs.reshape(rg * w_out, cin), w_taps[t],
                                    preferred_element_type=jnp.float32)
        elif fold:
            pr = [prow(r0 + i) for i in range(rg + 2)]
            lhs = jnp.concatenate(
                [jnp.concatenate([pr[i + dy][dx:dx + w_out]
                                  for dy in range(3) for dx in range(3)],
                                 axis=-1)
                 for i in range(rg)], axis=0)
            acc = jnp.dot(lhs, w_all, preferred_element_type=jnp.float32)
        else:
            pr = [prow(r0 + i) for i in range(rg + 2)]
            acc = jnp.zeros((rg * w_out, tco), jnp.float32)
            t = 0
            for dy in range(3):
                for dx in range(3):
                    if rg == 1:
                        l = pr[dy][dx:dx + w_out]
                    else:
                        l = jnp.concatenate(
                            [pr[i + dy][dx:dx + w_out] for i in range(rg)],
                            axis=0)
                    acc = acc + jnp.dot(l, w_taps[t],
                                        preferred_element_type=jnp.float32)
                    t += 1
        acc = acc + bias
        if emit_preact:
            f_ref[pl.ds(r0, rg)] = acc.reshape(rg, w_out, tco)
        y = jnp.maximum(acc, 0.0)
        if do_pool:
            y4 = y.reshape(rg // 2, 2, w_out, tco)
            m = jnp.maximum(y4[:, 0], y4[:, 1])
            m4 = m.reshape(rg // 2, w_out // 2, 2, tco)
            m = jnp.maximum(m4[:, :, 0, :], m4[:, :, 1, :])
            y_ref[pl.ds(r0 // 2, rg // 2)] = m.astype(y_ref.dtype)
        else:
            y_ref[pl.ds(r0, rg)] = y.reshape(rg, w_out, tco).astype(y_ref.dtype)


def _conv(x, w, b, *, preact, pool):
    n, h, wd, cin = x.shape
    cout = w.shape[-1]

    th = 8 if h % 8 == 0 else h
    n_rt = h // th
    tco = min(cout, 128)
    n_co = cout // tco
    pad_scr = (wd % 8 == 0) and n_rt > 1
    fold = (cin % 128 == 0) and not pad_scr

    rg = th
    for d in range(1, th + 1):
        if th % d == 0 and d * wd >= 112:
            rg = d
            break
    if pool and rg % 2:
        rg *= 2
    assert th % rg == 0 and (not pool or rg % 2 == 0)

    wb = w.astype(jnp.bfloat16)
    if fold:
        w_in = wb.reshape(9 * cin, cout)
        w_spec = pl.BlockSpec((9 * cin, tco), lambda i, j, k: (0, j))
    else:
        w_in = wb.reshape(9, cin, cout)
        w_spec = pl.BlockSpec((9, cin, tco), lambda i, j, k: (0, 0, j))
    b_in = b.reshape(1, cout)

    ho, wo = (h // 2, wd // 2) if pool else (h, wd)
    tho = th // 2 if pool else th
    y_sds = jax.ShapeDtypeStruct((n, ho, wo, cout), jnp.bfloat16)
    y_spec = pl.BlockSpec((None, tho, wo, tco), lambda i, j, k: (i, k, 0, j))
    if preact:
        f_sds = jax.ShapeDtypeStruct((n, h, wd, cout), jnp.float32)
        f_spec = pl.BlockSpec((None, th, wd, tco), lambda i, j, k: (i, k, 0, j))
        out_shape = (y_sds, f_sds)
        out_specs = (y_spec, f_spec)
    else:
        out_shape = y_sds
        out_specs = y_spec

    wp = wd + 16 if pad_scr else wd
    body = functools.partial(_conv_body, th=th, n_rt=n_rt, w_out=wd, cin=cin,
                             rg=rg, fold=fold, pad_scr=pad_scr,
                             emit_preact=preact, do_pool=pool)
    outs = pl.pallas_call(
        body,
        out_shape=out_shape,
        grid_spec=pltpu.PrefetchScalarGridSpec(
            num_scalar_prefetch=0,
            grid=(n, n_co, n_rt),
            in_specs=[
                pl.BlockSpec(memory_space=pl.ANY),
                w_spec,
                pl.BlockSpec((1, tco), lambda i, j, k: (0, j)),
            ],
            out_specs=out_specs,
            scratch_shapes=[
                pltpu.VMEM((th + 2, wp, cin), jnp.bfloat16) if n_rt == 1
                else pltpu.VMEM((2, th + 2, wp, cin), jnp.bfloat16),
                pltpu.SemaphoreType.DMA if n_rt == 1
                else pltpu.SemaphoreType.DMA((2,)),
            ]),
        compiler_params=pltpu.CompilerParams(
            dimension_semantics=("parallel", "parallel", "arbitrary")),
    )(x, w_in, b_in)
    if preact:
        return outs[0], outs[1]
    return outs, None


_PLAN = [(True, False), (False, True),
         (True, False), (False, True),
         (True, False), (False, False), (False, False), (False, True),
         (True, False), (False, False), (False, False), (False, True),
         (True, False)]


def kernel(x, w0, b0, w1, b1, w2, b2, w3, b3, w4, b4, w5, b5, w6, b6, w7, b7,
           w8, b8, w9, b9, w10, b10, w11, b11, w12, b12, w13, b13, w14, b14,
           w15, b15):
    ws = [w0, w1, w2, w3, w4, w5, w6, w7, w8, w9, w10, w11, w12]
    bs = [b0, b1, b2, b3, b4, b5, b6, b7, b8, b9, b10, b11, b12]
    x = jnp.transpose(x, (0, 2, 3, 1)).astype(jnp.bfloat16)
    feats = []
    for li, (pre, po) in enumerate(_PLAN):
        x, f = _conv(x, ws[li], bs[li], preact=pre, pool=po)
        if pre:
            feats.append(jnp.transpose(f, (0, 3, 1, 2)))
    return tuple(feats)


# speedup vs baseline: 1.0893x; 1.0842x over previous
import functools

import jax
import jax.numpy as jnp
from jax.experimental import pallas as pl
from jax.experimental.pallas import tpu as pltpu


def _conv_body(x_hbm, w_ref, b_ref, *refs, th, n_rt, w_out, cin, rg, fold,
               pad_scr, emit_preact, do_pool):
    if emit_preact:
        y_ref, f_ref, x_vmem, sem = refs
    else:
        y_ref, x_vmem, sem = refs
        f_ref = None

    n = pl.program_id(0)
    j = pl.program_id(1)
    rt = pl.program_id(2)
    dt = x_vmem.dtype
    wp = x_vmem.shape[-2]
    col0 = 8 if pad_scr else 0

    if n_rt == 1:
        @pl.when(j == 0)
        def _():
            x_vmem[0:1] = jnp.zeros((1, wp, cin), dt)
            x_vmem[th + 1:th + 2] = jnp.zeros((1, wp, cin), dt)
            cp = pltpu.make_async_copy(x_hbm.at[n], x_vmem.at[pl.ds(1, th)],
                                       sem)
            cp.start()
            cp.wait()

        def rows(a, m):
            return x_vmem[a:a + m]
    else:
        slot = jax.lax.rem(rt, 2)

        def dst(s, a, m):
            if pad_scr:
                return x_vmem.at[s, pl.ds(a, m), pl.ds(col0, w_out)]
            return x_vmem.at[s, pl.ds(a, m)]

        def halo(rt_t, s, mode):
            def go(cp):
                cp.start() if mode == 'start' else cp.wait()

            if isinstance(rt_t, int):
                assert rt_t == 0 and s == 0
                go(pltpu.make_async_copy(
                    x_hbm.at[n, pl.ds(0, th + 1)], dst(0, 1, th + 1),
                    sem.at[0]))
                return
            first = rt_t == 0
            last = rt_t == n_rt - 1
            r0_t = rt_t * th

            @pl.when(first)
            def _():
                go(pltpu.make_async_copy(
                    x_hbm.at[n, pl.ds(0, th + 1)], dst(s, 1, th + 1),
                    sem.at[s]))

            @pl.when(jnp.logical_and(jnp.logical_not(first),
                                     jnp.logical_not(last)))
            def _():
                go(pltpu.make_async_copy(
                    x_hbm.at[n, pl.ds(r0_t - 1, th + 2)], dst(s, 0, th + 2),
                    sem.at[s]))

            @pl.when(jnp.logical_and(last, jnp.logical_not(first)))
            def _():
                go(pltpu.make_async_copy(
                    x_hbm.at[n, pl.ds(r0_t - 1, th + 1)], dst(s, 0, th + 1),
                    sem.at[s]))

        @pl.when(rt == 0)
        def _():
            halo(0, 0, 'start')

        @pl.when(rt + 1 < n_rt)
        def _():
            halo(rt + 1, 1 - slot, 'start')

        halo(rt, slot, 'wait')

        @pl.when(rt == 0)
        def _():
            x_vmem[0, 0:1] = jnp.zeros((1, wp, cin), dt)

        @pl.when(rt == n_rt - 1)
        def _():
            x_vmem[(n_rt - 1) % 2, th + 1:th + 2] = jnp.zeros(
                (1, wp, cin), dt)

        if pad_scr:
            x_vmem[slot, :, col0 - 1:col0] = jnp.zeros((th + 2, 1, cin), dt)
            x_vmem[slot, :, col0 + w_out:col0 + w_out + 1] = jnp.zeros(
                (th + 2, 1, cin), dt)

        def rows(a, m):
            return x_vmem[slot, a:a + m]

    bias = b_ref[...]
    tco = b_ref.shape[-1]
    if fold:
        w_all = w_ref[...]
    else:
        w_taps = [w_ref[t] for t in range(9)]

    zcol = jnp.zeros((1, cin), dt)

    def prow(rr):
        return jnp.concatenate([zcol, rows(rr, 1).reshape(wp, cin), zcol],
                               axis=0)

    for g in range(th // rg):
        r0 = g * rg
        if pad_scr:
            acc = jnp.zeros((rg * w_out, tco), jnp.float32)
            for t, (dy, dx) in enumerate([(a, b) for a in range(3)
                                          for b in range(3)]):
                lhs = rows(r0 + dy, rg)[:, col0 - 1 + dx:col0 - 1 + dx + w_out]
                acc = acc + jnp.dot(lhs.reshape(rg * w_out, cin), w_taps[t],
                                    preferred_element_type=jnp.float32)
        elif fold:
            pr = [prow(r0 + i) for i in range(rg + 2)]
            lhs = jnp.concatenate(
                [jnp.concatenate([pr[i + dy][dx:dx + w_out]
                                  for dy in range(3) for dx in range(3)],
                                 axis=-1)
                 for i in range(rg)], axis=0)
            acc = jnp.dot(lhs, w_all, preferred_element_type=jnp.float32)
        else:
            pr = [prow(r0 + i) for i in range(rg + 2)]
            acc = jnp.zeros((rg * w_out, tco), jnp.float32)
            t = 0
            for dy in range(3):
                for dx in range(3):
                    if rg == 1:
                        l = pr[dy][dx:dx + w_out]
                    else:
                        l = jnp.concatenate(
                            [pr[i + dy][dx:dx + w_out] for i in range(rg)],
                            axis=0)
                    acc = acc + jnp.dot(l, w_taps[t],
                                        preferred_element_type=jnp.float32)
                    t += 1
        acc = acc + bias
        if emit_preact:
            f_ref[pl.ds(r0, rg)] = acc.reshape(rg, w_out, tco)
        y = jnp.maximum(acc, 0.0)
        if do_pool:
            y4 = y.reshape(rg // 2, 2, w_out, tco)
            m = jnp.maximum(y4[:, 0], y4[:, 1])
            m4 = m.reshape(rg // 2, w_out // 2, 2, tco)
            m = jnp.maximum(m4[:, :, 0, :], m4[:, :, 1, :])
            y_ref[pl.ds(r0 // 2, rg // 2)] = m.astype(y_ref.dtype)
        else:
            y_ref[pl.ds(r0, rg)] = y.reshape(rg, w_out, tco).astype(y_ref.dtype)


def _conv(x, w, b, *, preact, pool):
    n, h, wd, cin = x.shape
    cout = w.shape[-1]

    th = 8 if h % 8 == 0 else h
    n_rt = h // th
    tco = min(cout, 128)
    n_co = cout // tco
    pad_scr = False
    fold = (cin % 128 == 0) and not pad_scr

    rg = th
    for d in range(1, th + 1):
        if th % d == 0 and d * wd >= 112:
            rg = d
            break
    if pool and rg % 2:
        rg *= 2
    assert th % rg == 0 and (not pool or rg % 2 == 0)

    wb = w.astype(jnp.bfloat16)
    if fold:
        w_in = wb.reshape(9 * cin, cout)
        w_spec = pl.BlockSpec((9 * cin, tco), lambda i, j, k: (0, j))
    else:
        w_in = wb.reshape(9, cin, cout)
        w_spec = pl.BlockSpec((9, cin, tco), lambda i, j, k: (0, 0, j))
    b_in = b.reshape(1, cout)

    ho, wo = (h // 2, wd // 2) if pool else (h, wd)
    tho = th // 2 if pool else th
    y_sds = jax.ShapeDtypeStruct((n, ho, wo, cout), jnp.bfloat16)
    y_spec = pl.BlockSpec((None, tho, wo, tco), lambda i, j, k: (i, k, 0, j))
    if preact:
        f_sds = jax.ShapeDtypeStruct((n, h, wd, cout), jnp.float32)
        f_spec = pl.BlockSpec((None, th, wd, tco), lambda i, j, k: (i, k, 0, j))
        out_shape = (y_sds, f_sds)
        out_specs = (y_spec, f_spec)
    else:
        out_shape = y_sds
        out_specs = y_spec

    wp = wd + 16 if pad_scr else wd
    body = functools.partial(_conv_body, th=th, n_rt=n_rt, w_out=wd, cin=cin,
                             rg=rg, fold=fold, pad_scr=pad_scr,
                             emit_preact=preact, do_pool=pool)
    outs = pl.pallas_call(
        body,
        out_shape=out_shape,
        grid_spec=pltpu.PrefetchScalarGridSpec(
            num_scalar_prefetch=0,
            grid=(n, n_co, n_rt),
            in_specs=[
                pl.BlockSpec(memory_space=pl.ANY),
                w_spec,
                pl.BlockSpec((1, tco), lambda i, j, k: (0, j)),
            ],
            out_specs=out_specs,
            scratch_shapes=[
                pltpu.VMEM((th + 2, wp, cin), jnp.bfloat16) if n_rt == 1
                else pltpu.VMEM((2, th + 2, wp, cin), jnp.bfloat16),
                pltpu.SemaphoreType.DMA if n_rt == 1
                else pltpu.SemaphoreType.DMA((2,)),
            ]),
        compiler_params=pltpu.CompilerParams(
            dimension_semantics=("parallel", "parallel", "arbitrary")),
    )(x, w_in, b_in)
    if preact:
        return outs[0], outs[1]
    return outs, None


_PLAN = [(True, False), (False, True),
         (True, False), (False, True),
         (True, False), (False, False), (False, False), (False, True),
         (True, False), (False, False), (False, False), (False, True),
         (True, False)]


def kernel(x, w0, b0, w1, b1, w2, b2, w3, b3, w4, b4, w5, b5, w6, b6, w7, b7,
           w8, b8, w9, b9, w10, b10, w11, b11, w12, b12, w13, b13, w14, b14,
           w15, b15):
    ws = [w0, w1, w2, w3, w4, w5, w6, w7, w8, w9, w10, w11, w12]
    bs = [b0, b1, b2, b3, b4, b5, b6, b7, b8, b9, b10, b11, b12]
    x = jnp.transpose(x, (0, 2, 3, 1)).astype(jnp.bfloat16)
    feats = []
    for li, (pre, po) in enumerate(_PLAN):
        x, f = _conv(x, ws[li], bs[li], preact=pre, pool=po)
        if pre:
            feats.append(jnp.transpose(f, (0, 3, 1, 2)))
    return tuple(feats)


# speedup vs baseline: 1.1351x; 1.0420x over previous
import functools

import jax
import jax.numpy as jnp
from jax.experimental import pallas as pl
from jax.experimental.pallas import tpu as pltpu


def _conv_body(x_hbm, w_ref, b_ref, *refs, th, n_rt, w_out, cin, rg, fold,
               pad_scr, emit_preact, do_pool):
    if emit_preact:
        y_ref, f_ref, x_vmem, sem = refs
    else:
        y_ref, x_vmem, sem = refs
        f_ref = None

    n = pl.program_id(0)
    j = pl.program_id(1)
    rt = pl.program_id(2)
    dt = x_vmem.dtype
    wp = x_vmem.shape[-2]
    col0 = 8 if pad_scr else 0

    if n_rt == 1:
        @pl.when(j == 0)
        def _():
            x_vmem[0:1] = jnp.zeros((1, wp, cin), dt)
            x_vmem[th + 1:th + 2] = jnp.zeros((1, wp, cin), dt)
            cp = pltpu.make_async_copy(x_hbm.at[n], x_vmem.at[pl.ds(1, th)],
                                       sem)
            cp.start()
            cp.wait()

        def rows(a, m):
            return x_vmem[a:a + m]
    else:
        slot = jax.lax.rem(rt, 2)

        def dst(s, a, m):
            if pad_scr:
                return x_vmem.at[s, pl.ds(a, m), pl.ds(col0, w_out)]
            return x_vmem.at[s, pl.ds(a, m)]

        def halo(rt_t, s, mode):
            def go(cp):
                cp.start() if mode == 'start' else cp.wait()

            if isinstance(rt_t, int):
                assert rt_t == 0 and s == 0
                go(pltpu.make_async_copy(
                    x_hbm.at[n, pl.ds(0, th + 1)], dst(0, 1, th + 1),
                    sem.at[0]))
                return
            first = rt_t == 0
            last = rt_t == n_rt - 1
            r0_t = rt_t * th

            @pl.when(first)
            def _():
                go(pltpu.make_async_copy(
                    x_hbm.at[n, pl.ds(0, th + 1)], dst(s, 1, th + 1),
                    sem.at[s]))

            @pl.when(jnp.logical_and(jnp.logical_not(first),
                                     jnp.logical_not(last)))
            def _():
                go(pltpu.make_async_copy(
                    x_hbm.at[n, pl.ds(r0_t - 1, th + 2)], dst(s, 0, th + 2),
                    sem.at[s]))

            @pl.when(jnp.logical_and(last, jnp.logical_not(first)))
            def _():
                go(pltpu.make_async_copy(
                    x_hbm.at[n, pl.ds(r0_t - 1, th + 1)], dst(s, 0, th + 1),
                    sem.at[s]))

        @pl.when(rt == 0)
        def _():
            halo(0, 0, 'start')

        @pl.when(rt + 1 < n_rt)
        def _():
            halo(rt + 1, 1 - slot, 'start')

        halo(rt, slot, 'wait')

        @pl.when(rt == 0)
        def _():
            x_vmem[0, 0:1] = jnp.zeros((1, wp, cin), dt)

        @pl.when(rt == n_rt - 1)
        def _():
            x_vmem[(n_rt - 1) % 2, th + 1:th + 2] = jnp.zeros(
                (1, wp, cin), dt)

        if pad_scr:
            x_vmem[slot, :, col0 - 1:col0] = jnp.zeros((th + 2, 1, cin), dt)
            x_vmem[slot, :, col0 + w_out:col0 + w_out + 1] = jnp.zeros(
                (th + 2, 1, cin), dt)

        def rows(a, m):
            return x_vmem[slot, a:a + m]

    bias = b_ref[...]
    tco = b_ref.shape[-1]
    if fold:
        w_all = w_ref[...]
    else:
        w_taps = [w_ref[t] for t in range(9)]

    zcol = jnp.zeros((1, cin), dt)

    def prow(rr):
        return jnp.concatenate([zcol, rows(rr, 1).reshape(wp, cin), zcol],
                               axis=0)

    for g in range(th // rg):
        r0 = g * rg
        if pad_scr:
            acc = jnp.zeros((rg * w_out, tco), jnp.float32)
            for t, (dy, dx) in enumerate([(a, b) for a in range(3)
                                          for b in range(3)]):
                lhs = rows(r0 + dy, rg)[:, col0 - 1 + dx:col0 - 1 + dx + w_out]
                acc = acc + jnp.dot(lhs.reshape(rg * w_out, cin), w_taps[t],
                                    preferred_element_type=jnp.float32)
        elif fold:
            pr = [prow(r0 + i) for i in range(rg + 2)]
            lhs = jnp.concatenate(
                [jnp.concatenate([pr[i + dy][dx:dx + w_out]
                                  for dy in range(3) for dx in range(3)],
                                 axis=-1)
                 for i in range(rg)], axis=0)
            acc = jnp.dot(lhs, w_all, preferred_element_type=jnp.float32)
        else:
            pr = [prow(r0 + i) for i in range(rg + 2)]
            acc = jnp.zeros((rg * w_out, tco), jnp.float32)
            t = 0
            for dy in range(3):
                for dx in range(3):
                    if rg == 1:
                        l = pr[dy][dx:dx + w_out]
                    else:
                        l = jnp.concatenate(
                            [pr[i + dy][dx:dx + w_out] for i in range(rg)],
                            axis=0)
                    acc = acc + jnp.dot(l, w_taps[t],
                                        preferred_element_type=jnp.float32)
                    t += 1
        acc = acc + bias
        if emit_preact:
            f_ref[pl.ds(r0, rg)] = acc.reshape(rg, w_out, tco)
        y = jnp.maximum(acc, 0.0)
        if do_pool:
            y4 = y.reshape(rg // 2, 2, w_out, tco)
            m = jnp.maximum(y4[:, 0], y4[:, 1])
            m4 = m.reshape(rg // 2, w_out // 2, 2, tco)
            m = jnp.maximum(m4[:, :, 0, :], m4[:, :, 1, :])
            y_ref[pl.ds(r0 // 2, rg // 2)] = m.astype(y_ref.dtype)
        else:
            y_ref[pl.ds(r0, rg)] = y.reshape(rg, w_out, tco).astype(y_ref.dtype)


def _conv(x, w, b, *, preact, pool):
    n, h, wd, cin = x.shape
    cout = w.shape[-1]

    th = 16 if h % 16 == 0 else (8 if h % 8 == 0 else h)
    n_rt = h // th
    tco = min(cout, 128)
    n_co = cout // tco
    pad_scr = False
    fold = (cin % 128 == 0) and not pad_scr

    rg = th
    for d in range(1, th + 1):
        if th % d == 0 and d * wd >= 112:
            rg = d
            break
    if pool and rg % 2:
        rg *= 2
    assert th % rg == 0 and (not pool or rg % 2 == 0)

    wb = w.astype(jnp.bfloat16)
    if fold:
        w_in = wb.reshape(9 * cin, cout)
        w_spec = pl.BlockSpec((9 * cin, tco), lambda i, j, k: (0, j))
    else:
        w_in = wb.reshape(9, cin, cout)
        w_spec = pl.BlockSpec((9, cin, tco), lambda i, j, k: (0, 0, j))
    b_in = b.reshape(1, cout)

    ho, wo = (h // 2, wd // 2) if pool else (h, wd)
    tho = th // 2 if pool else th
    y_sds = jax.ShapeDtypeStruct((n, ho, wo, cout), jnp.bfloat16)
    y_spec = pl.BlockSpec((None, tho, wo, tco), lambda i, j, k: (i, k, 0, j))
    if preact:
        f_sds = jax.ShapeDtypeStruct((n, h, wd, cout), jnp.float32)
        f_spec = pl.BlockSpec((None, th, wd, tco), lambda i, j, k: (i, k, 0, j))
        out_shape = (y_sds, f_sds)
        out_specs = (y_spec, f_spec)
    else:
        out_shape = y_sds
        out_specs = y_spec

    wp = wd + 16 if pad_scr else wd
    body = functools.partial(_conv_body, th=th, n_rt=n_rt, w_out=wd, cin=cin,
                             rg=rg, fold=fold, pad_scr=pad_scr,
                             emit_preact=preact, do_pool=pool)
    outs = pl.pallas_call(
        body,
        out_shape=out_shape,
        grid_spec=pltpu.PrefetchScalarGridSpec(
            num_scalar_prefetch=0,
            grid=(n, n_co, n_rt),
            in_specs=[
                pl.BlockSpec(memory_space=pl.ANY),
                w_spec,
                pl.BlockSpec((1, tco), lambda i, j, k: (0, j)),
            ],
            out_specs=out_specs,
            scratch_shapes=[
                pltpu.VMEM((th + 2, wp, cin), jnp.bfloat16) if n_rt == 1
                else pltpu.VMEM((2, th + 2, wp, cin), jnp.bfloat16),
                pltpu.SemaphoreType.DMA if n_rt == 1
                else pltpu.SemaphoreType.DMA((2,)),
            ]),
        compiler_params=pltpu.CompilerParams(
            dimension_semantics=("parallel", "parallel", "arbitrary")),
    )(x, w_in, b_in)
    if preact:
        return outs[0], outs[1]
    return outs, None


_PLAN = [(True, False), (False, True),
         (True, False), (False, True),
         (True, False), (False, False), (False, False), (False, True),
         (True, False), (False, False), (False, False), (False, True),
         (True, False)]


def kernel(x, w0, b0, w1, b1, w2, b2, w3, b3, w4, b4, w5, b5, w6, b6, w7, b7,
           w8, b8, w9, b9, w10, b10, w11, b11, w12, b12, w13, b13, w14, b14,
           w15, b15):
    ws = [w0, w1, w2, w3, w4, w5, w6, w7, w8, w9, w10, w11, w12]
    bs = [b0, b1, b2, b3, b4, b5, b6, b7, b8, b9, b10, b11, b12]
    x = jnp.transpose(x, (0, 2, 3, 1)).astype(jnp.bfloat16)
    feats = []
    for li, (pre, po) in enumerate(_PLAN):
        x, f = _conv(x, ws[li], bs[li], preact=pre, pool=po)
        if pre:
            feats.append(jnp.transpose(f, (0, 3, 1, 2)))
    return tuple(feats)


# speedup vs baseline: 1.2240x; 1.0784x over previous
import functools

import jax
import jax.numpy as jnp
from jax.experimental import pallas as pl
from jax.experimental.pallas import tpu as pltpu


def _conv_body(x_hbm, w_ref, b_ref, *refs, th, n_rt, w_out, cin, rg, fold,
               pad_scr, emit_preact, do_pool):
    if emit_preact:
        y_ref, f_ref, x_vmem, sem = refs
    else:
        y_ref, x_vmem, sem = refs
        f_ref = None

    n = pl.program_id(0)
    j = pl.program_id(1)
    rt = pl.program_id(2)
    dt = x_vmem.dtype
    wp = x_vmem.shape[-2]
    col0 = 8 if pad_scr else 0

    if n_rt == 1:
        @pl.when(j == 0)
        def _():
            x_vmem[0:1] = jnp.zeros((1, wp, cin), dt)
            x_vmem[th + 1:th + 2] = jnp.zeros((1, wp, cin), dt)
            cp = pltpu.make_async_copy(x_hbm.at[n], x_vmem.at[pl.ds(1, th)],
                                       sem)
            cp.start()
            cp.wait()

        def rows(a, m):
            return x_vmem[a:a + m]
    else:
        slot = jax.lax.rem(rt, 2)

        def dst(s, a, m):
            if pad_scr:
                return x_vmem.at[s, pl.ds(a, m), pl.ds(col0, w_out)]
            return x_vmem.at[s, pl.ds(a, m)]

        def halo(rt_t, s, mode):
            def go(cp):
                cp.start() if mode == 'start' else cp.wait()

            if isinstance(rt_t, int):
                assert rt_t == 0 and s == 0
                go(pltpu.make_async_copy(
                    x_hbm.at[n, pl.ds(0, th + 1)], dst(0, 1, th + 1),
                    sem.at[0]))
                return
            first = rt_t == 0
            last = rt_t == n_rt - 1
            r0_t = rt_t * th

            @pl.when(first)
            def _():
                go(pltpu.make_async_copy(
                    x_hbm.at[n, pl.ds(0, th + 1)], dst(s, 1, th + 1),
                    sem.at[s]))

            @pl.when(jnp.logical_and(jnp.logical_not(first),
                                     jnp.logical_not(last)))
            def _():
                go(pltpu.make_async_copy(
                    x_hbm.at[n, pl.ds(r0_t - 1, th + 2)], dst(s, 0, th + 2),
                    sem.at[s]))

            @pl.when(jnp.logical_and(last, jnp.logical_not(first)))
            def _():
                go(pltpu.make_async_copy(
                    x_hbm.at[n, pl.ds(r0_t - 1, th + 1)], dst(s, 0, th + 1),
                    sem.at[s]))

        @pl.when(rt == 0)
        def _():
            halo(0, 0, 'start')

        @pl.when(rt + 1 < n_rt)
        def _():
            halo(rt + 1, 1 - slot, 'start')

        halo(rt, slot, 'wait')

        @pl.when(rt == 0)
        def _():
            x_vmem[0, 0:1] = jnp.zeros((1, wp, cin), dt)

        @pl.when(rt == n_rt - 1)
        def _():
            x_vmem[(n_rt - 1) % 2, th + 1:th + 2] = jnp.zeros(
                (1, wp, cin), dt)

        if pad_scr:
            x_vmem[slot, :, col0 - 1:col0] = jnp.zeros((th + 2, 1, cin), dt)
            x_vmem[slot, :, col0 + w_out:col0 + w_out + 1] = jnp.zeros(
                (th + 2, 1, cin), dt)

        def rows(a, m):
            return x_vmem[slot, a:a + m]

    bias = b_ref[...]
    tco = b_ref.shape[-1]
    if fold:
        w_all = w_ref[...]
    else:
        w_taps = [w_ref[t] for t in range(9)]

    zcol = jnp.zeros((1, cin), dt)

    def prow(rr):
        return jnp.concatenate([zcol, rows(rr, 1).reshape(wp, cin), zcol],
                               axis=0)

    for g in range(th // rg):
        r0 = g * rg
        if pad_scr:
            acc = jnp.zeros((rg * w_out, tco), jnp.float32)
            for t, (dy, dx) in enumerate([(a, b) for a in range(3)
                                          for b in range(3)]):
                lhs = rows(r0 + dy, rg)[:, col0 - 1 + dx:col0 - 1 + dx + w_out]
                acc = acc + jnp.dot(lhs.reshape(rg * w_out, cin), w_taps[t],
                                    preferred_element_type=jnp.float32)
        elif fold:
            pr = [prow(r0 + i) for i in range(rg + 2)]
            lhs = jnp.concatenate(
                [jnp.concatenate([pr[i + dy][dx:dx + w_out]
                                  for dy in range(3) for dx in range(3)],
                                 axis=-1)
                 for i in range(rg)], axis=0)
            acc = jnp.dot(lhs, w_all, preferred_element_type=jnp.float32)
        else:
            pr = [prow(r0 + i) for i in range(rg + 2)]
            acc = jnp.zeros((rg * w_out, tco), jnp.float32)
            t = 0
            for dy in range(3):
                for dx in range(3):
                    if rg == 1:
                        l = pr[dy][dx:dx + w_out]
                    else:
                        l = jnp.concatenate(
                            [pr[i + dy][dx:dx + w_out] for i in range(rg)],
                            axis=0)
                    acc = acc + jnp.dot(l, w_taps[t],
                                        preferred_element_type=jnp.float32)
                    t += 1
        acc = acc + bias
        if emit_preact:
            f_ref[pl.ds(r0, rg)] = acc.reshape(rg, w_out, tco)
        y = jnp.maximum(acc, 0.0)
        if do_pool:
            y4 = y.reshape(rg // 2, 2, w_out, tco)
            m = jnp.maximum(y4[:, 0], y4[:, 1])
            m4 = m.reshape(rg // 2, w_out // 2, 2, tco)
            m = jnp.maximum(m4[:, :, 0, :], m4[:, :, 1, :])
            y_ref[pl.ds(r0 // 2, rg // 2)] = m.astype(y_ref.dtype)
        else:
            y_ref[pl.ds(r0, rg)] = y.reshape(rg, w_out, tco).astype(y_ref.dtype)


def _conv(x, w, b, *, preact, pool):
    n, h, wd, cin = x.shape
    cout = w.shape[-1]

    th = h
    for t in (32, 28, 16, 8):
        if h % t == 0:
            th = t
            break
    n_rt = h // th
    tco = min(cout, 128)
    n_co = cout // tco
    pad_scr = False
    fold = (cin % 128 == 0) and not pad_scr

    rg = th
    for d in range(1, th + 1):
        if th % d == 0 and d * wd >= 112:
            rg = d
            break
    if pool and rg % 2:
        rg *= 2
    assert th % rg == 0 and (not pool or rg % 2 == 0)

    wb = w.astype(jnp.bfloat16)
    if fold:
        w_in = wb.reshape(9 * cin, cout)
        w_spec = pl.BlockSpec((9 * cin, tco), lambda i, j, k: (0, j))
    else:
        w_in = wb.reshape(9, cin, cout)
        w_spec = pl.BlockSpec((9, cin, tco), lambda i, j, k: (0, 0, j))
    b_in = b.reshape(1, cout)

    ho, wo = (h // 2, wd // 2) if pool else (h, wd)
    tho = th // 2 if pool else th
    y_sds = jax.ShapeDtypeStruct((n, ho, wo, cout), jnp.bfloat16)
    y_spec = pl.BlockSpec((None, tho, wo, tco), lambda i, j, k: (i, k, 0, j))
    if preact:
        f_sds = jax.ShapeDtypeStruct((n, h, wd, cout), jnp.float32)
        f_spec = pl.BlockSpec((None, th, wd, tco), lambda i, j, k: (i, k, 0, j))
        out_shape = (y_sds, f_sds)
        out_specs = (y_spec, f_spec)
    else:
        out_shape = y_sds
        out_specs = y_spec

    wp = wd + 16 if pad_scr else wd
    body = functools.partial(_conv_body, th=th, n_rt=n_rt, w_out=wd, cin=cin,
                             rg=rg, fold=fold, pad_scr=pad_scr,
                             emit_preact=preact, do_pool=pool)
    outs = pl.pallas_call(
        body,
        out_shape=out_shape,
        grid_spec=pltpu.PrefetchScalarGridSpec(
            num_scalar_prefetch=0,
            grid=(n, n_co, n_rt),
            in_specs=[
                pl.BlockSpec(memory_space=pl.ANY),
                w_spec,
                pl.BlockSpec((1, tco), lambda i, j, k: (0, j)),
            ],
            out_specs=out_specs,
            scratch_shapes=[
                pltpu.VMEM((th + 2, wp, cin), jnp.bfloat16) if n_rt == 1
                else pltpu.VMEM((2, th + 2, wp, cin), jnp.bfloat16),
                pltpu.SemaphoreType.DMA if n_rt == 1
                else pltpu.SemaphoreType.DMA((2,)),
            ]),
        compiler_params=pltpu.CompilerParams(
            dimension_semantics=("parallel", "parallel", "arbitrary")),
    )(x, w_in, b_in)
    if preact:
        return outs[0], outs[1]
    return outs, None


_PLAN = [(True, False), (False, True),
         (True, False), (False, True),
         (True, False), (False, False), (False, False), (False, True),
         (True, False), (False, False), (False, False), (False, True),
         (True, False)]


def kernel(x, w0, b0, w1, b1, w2, b2, w3, b3, w4, b4, w5, b5, w6, b6, w7, b7,
           w8, b8, w9, b9, w10, b10, w11, b11, w12, b12, w13, b13, w14, b14,
           w15, b15):
    ws = [w0, w1, w2, w3, w4, w5, w6, w7, w8, w9, w10, w11, w12]
    bs = [b0, b1, b2, b3, b4, b5, b6, b7, b8, b9, b10, b11, b12]
    x = jnp.transpose(x, (0, 2, 3, 1)).astype(jnp.bfloat16)
    feats = []
    for li, (pre, po) in enumerate(_PLAN):
        x, f = _conv(x, ws[li], bs[li], preact=pre, pool=po)
        if pre:
            feats.append(jnp.transpose(f, (0, 3, 1, 2)))
    return tuple(feats)


# speedup vs baseline: 1.2570x; 1.0269x over previous
import functools

import jax
import jax.numpy as jnp
from jax.experimental import pallas as pl
from jax.experimental.pallas import tpu as pltpu


def _conv_body(x_hbm, w_ref, b_ref, *refs, th, n_rt, w_out, cin, rg, fold,
               pad_scr, emit_preact, do_pool):
    if emit_preact:
        y_ref, f_ref, x_vmem, sem = refs
    else:
        y_ref, x_vmem, sem = refs
        f_ref = None

    n = pl.program_id(0)
    j = pl.program_id(1)
    rt = pl.program_id(2)
    dt = x_vmem.dtype
    wp = x_vmem.shape[-2]
    col0 = 8 if pad_scr else 0

    if n_rt == 1:
        @pl.when(j == 0)
        def _():
            x_vmem[0:1] = jnp.zeros((1, wp, cin), dt)
            x_vmem[th + 1:th + 2] = jnp.zeros((1, wp, cin), dt)
            cp = pltpu.make_async_copy(x_hbm.at[n], x_vmem.at[pl.ds(1, th)],
                                       sem)
            cp.start()
            cp.wait()

        def rows(a, m):
            return x_vmem[a:a + m]
    else:
        slot = jax.lax.rem(rt, 2)

        def dst(s, a, m):
            if pad_scr:
                return x_vmem.at[s, pl.ds(a, m), pl.ds(col0, w_out)]
            return x_vmem.at[s, pl.ds(a, m)]

        def halo(rt_t, s, mode):
            def go(cp):
                cp.start() if mode == 'start' else cp.wait()

            if isinstance(rt_t, int):
                assert rt_t == 0 and s == 0
                go(pltpu.make_async_copy(
                    x_hbm.at[n, pl.ds(0, th + 1)], dst(0, 1, th + 1),
                    sem.at[0]))
                return
            first = rt_t == 0
            last = rt_t == n_rt - 1
            r0_t = rt_t * th

            @pl.when(first)
            def _():
                go(pltpu.make_async_copy(
                    x_hbm.at[n, pl.ds(0, th + 1)], dst(s, 1, th + 1),
                    sem.at[s]))

            @pl.when(jnp.logical_and(jnp.logical_not(first),
                                     jnp.logical_not(last)))
            def _():
                go(pltpu.make_async_copy(
                    x_hbm.at[n, pl.ds(r0_t - 1, th + 2)], dst(s, 0, th + 2),
                    sem.at[s]))

            @pl.when(jnp.logical_and(last, jnp.logical_not(first)))
            def _():
                go(pltpu.make_async_copy(
                    x_hbm.at[n, pl.ds(r0_t - 1, th + 1)], dst(s, 0, th + 1),
                    sem.at[s]))

        @pl.when(rt == 0)
        def _():
            halo(0, 0, 'start')

        @pl.when(rt + 1 < n_rt)
        def _():
            halo(rt + 1, 1 - slot, 'start')

        halo(rt, slot, 'wait')

        @pl.when(rt == 0)
        def _():
            x_vmem[0, 0:1] = jnp.zeros((1, wp, cin), dt)

        @pl.when(rt == n_rt - 1)
        def _():
            x_vmem[(n_rt - 1) % 2, th + 1:th + 2] = jnp.zeros(
                (1, wp, cin), dt)

        if pad_scr:
            x_vmem[slot, :, col0 - 1:col0] = jnp.zeros((th + 2, 1, cin), dt)
            x_vmem[slot, :, col0 + w_out:col0 + w_out + 1] = jnp.zeros(
                (th + 2, 1, cin), dt)

        def rows(a, m):
            return x_vmem[slot, a:a + m]

    bias = b_ref[...]
    tco = b_ref.shape[-1]
    if fold:
        w_all = w_ref[...]
    else:
        w_taps = [w_ref[t] for t in range(9)]

    zcol = jnp.zeros((1, cin), dt)

    def prow(rr):
        return jnp.concatenate([zcol, rows(rr, 1).reshape(wp, cin), zcol],
                               axis=0)

    for g in range(th // rg):
        r0 = g * rg
        if pad_scr:
            acc = jnp.zeros((rg * w_out, tco), jnp.float32)
            for t, (dy, dx) in enumerate([(a, b) for a in range(3)
                                          for b in range(3)]):
                lhs = rows(r0 + dy, rg)[:, col0 - 1 + dx:col0 - 1 + dx + w_out]
                acc = acc + jnp.dot(lhs.reshape(rg * w_out, cin), w_taps[t],
                                    preferred_element_type=jnp.float32)
        elif fold:
            pr = [prow(r0 + i) for i in range(rg + 2)]
            lhs = jnp.concatenate(
                [jnp.concatenate([pr[i + dy][dx:dx + w_out]
                                  for dy in range(3) for dx in range(3)],
                                 axis=-1)
                 for i in range(rg)], axis=0)
            acc = jnp.dot(lhs, w_all, preferred_element_type=jnp.float32)
        else:
            pr = [prow(r0 + i) for i in range(rg + 2)]
            acc = jnp.zeros((rg * w_out, tco), jnp.float32)
            t = 0
            for dy in range(3):
                for dx in range(3):
                    if rg == 1:
                        l = pr[dy][dx:dx + w_out]
                    else:
                        l = jnp.concatenate(
                            [pr[i + dy][dx:dx + w_out] for i in range(rg)],
                            axis=0)
                    acc = acc + jnp.dot(l, w_taps[t],
                                        preferred_element_type=jnp.float32)
                    t += 1
        acc = acc + bias
        if emit_preact:
            f_ref[pl.ds(r0, rg)] = acc.reshape(rg, w_out, tco)
        y = jnp.maximum(acc, 0.0)
        if do_pool:
            y4 = y.reshape(rg // 2, 2, w_out, tco)
            m = jnp.maximum(y4[:, 0], y4[:, 1])
            m4 = m.reshape(rg // 2, w_out // 2, 2, tco)
            m = jnp.maximum(m4[:, :, 0, :], m4[:, :, 1, :])
            y_ref[pl.ds(r0 // 2, rg // 2)] = m.astype(y_ref.dtype)
        else:
            y_ref[pl.ds(r0, rg)] = y.reshape(rg, w_out, tco).astype(y_ref.dtype)


def _conv(x, w, b, *, preact, pool):
    n, h, wd, cin = x.shape
    cout = w.shape[-1]

    th = h
    for t in (56, 32, 28, 16, 8):
        if h % t == 0:
            th = t
            break
    n_rt = h // th
    tco = min(cout, 128)
    n_co = cout // tco
    pad_scr = False
    fold = (cin % 128 == 0) and not pad_scr

    rg = th
    for d in range(1, th + 1):
        if th % d == 0 and d * wd >= 112:
            rg = d
            break
    if pool and rg % 2:
        rg *= 2
    assert th % rg == 0 and (not pool or rg % 2 == 0)

    wb = w.astype(jnp.bfloat16)
    if fold:
        w_in = wb.reshape(9 * cin, cout)
        w_spec = pl.BlockSpec((9 * cin, tco), lambda i, j, k: (0, j))
    else:
        w_in = wb.reshape(9, cin, cout)
        w_spec = pl.BlockSpec((9, cin, tco), lambda i, j, k: (0, 0, j))
    b_in = b.reshape(1, cout)

    ho, wo = (h // 2, wd // 2) if pool else (h, wd)
    tho = th // 2 if pool else th
    y_sds = jax.ShapeDtypeStruct((n, ho, wo, cout), jnp.bfloat16)
    y_spec = pl.BlockSpec((None, tho, wo, tco), lambda i, j, k: (i, k, 0, j))
    if preact:
        f_sds = jax.ShapeDtypeStruct((n, h, wd, cout), jnp.float32)
        f_spec = pl.BlockSpec((None, th, wd, tco), lambda i, j, k: (i, k, 0, j))
        out_shape = (y_sds, f_sds)
        out_specs = (y_spec, f_spec)
    else:
        out_shape = y_sds
        out_specs = y_spec

    wp = wd + 16 if pad_scr else wd
    body = functools.partial(_conv_body, th=th, n_rt=n_rt, w_out=wd, cin=cin,
                             rg=rg, fold=fold, pad_scr=pad_scr,
                             emit_preact=preact, do_pool=pool)
    outs = pl.pallas_call(
        body,
        out_shape=out_shape,
        grid_spec=pltpu.PrefetchScalarGridSpec(
            num_scalar_prefetch=0,
            grid=(n, n_co, n_rt),
            in_specs=[
                pl.BlockSpec(memory_space=pl.ANY),
                w_spec,
                pl.BlockSpec((1, tco), lambda i, j, k: (0, j)),
            ],
            out_specs=out_specs,
            scratch_shapes=[
                pltpu.VMEM((th + 2, wp, cin), jnp.bfloat16) if n_rt == 1
                else pltpu.VMEM((2, th + 2, wp, cin), jnp.bfloat16),
                pltpu.SemaphoreType.DMA if n_rt == 1
                else pltpu.SemaphoreType.DMA((2,)),
            ]),
        compiler_params=pltpu.CompilerParams(
            dimension_semantics=("parallel", "parallel", "arbitrary")),
    )(x, w_in, b_in)
    if preact:
        return outs[0], outs[1]
    return outs, None


_PLAN = [(True, False), (False, True),
         (True, False), (False, True),
         (True, False), (False, False), (False, False), (False, True),
         (True, False), (False, False), (False, False), (False, True),
         (True, False)]


def kernel(x, w0, b0, w1, b1, w2, b2, w3, b3, w4, b4, w5, b5, w6, b6, w7, b7,
           w8, b8, w9, b9, w10, b10, w11, b11, w12, b12, w13, b13, w14, b14,
           w15, b15):
    ws = [w0, w1, w2, w3, w4, w5, w6, w7, w8, w9, w10, w11, w12]
    bs = [b0, b1, b2, b3, b4, b5, b6, b7, b8, b9, b10, b11, b12]
    x = jnp.transpose(x, (0, 2, 3, 1)).astype(jnp.bfloat16)
    feats = []
    for li, (pre, po) in enumerate(_PLAN):
        x, f = _conv(x, ws[li], bs[li], preact=pre, pool=po)
        if pre:
            feats.append(jnp.transpose(f, (0, 3, 1, 2)))
    return tuple(feats)


# speedup vs baseline: 1.3533x; 1.0767x over previous
import functools

import jax
import jax.numpy as jnp
from jax.experimental import pallas as pl
from jax.experimental.pallas import tpu as pltpu


def _conv_body(x_hbm, w_ref, b_ref, *refs, th, n_rt, w_out, cin, rg, fold,
               pad_scr, dy_only, emit_preact, do_pool):
    if emit_preact:
        y_ref, f_ref, x_vmem, sem = refs
    else:
        y_ref, x_vmem, sem = refs
        f_ref = None

    n = pl.program_id(0)
    j = pl.program_id(1)
    rt = pl.program_id(2)
    dt = x_vmem.dtype
    wp = x_vmem.shape[-2]
    col0 = 8 if pad_scr else 0

    if n_rt == 1:
        @pl.when(j == 0)
        def _():
            x_vmem[0:1] = jnp.zeros((1, wp, cin), dt)
            x_vmem[th + 1:th + 2] = jnp.zeros((1, wp, cin), dt)
            cp = pltpu.make_async_copy(x_hbm.at[n], x_vmem.at[pl.ds(1, th)],
                                       sem)
            cp.start()
            cp.wait()

        def rows(a, m):
            return x_vmem[a:a + m]
    else:
        slot = jax.lax.rem(rt, 2)

        def dst(s, a, m):
            if pad_scr:
                return x_vmem.at[s, pl.ds(a, m), pl.ds(col0, w_out)]
            return x_vmem.at[s, pl.ds(a, m)]

        def halo(rt_t, s, mode):
            def go(cp):
                cp.start() if mode == 'start' else cp.wait()

            if isinstance(rt_t, int):
                assert rt_t == 0 and s == 0
                go(pltpu.make_async_copy(
                    x_hbm.at[n, pl.ds(0, th + 1)], dst(0, 1, th + 1),
                    sem.at[0]))
                return
            first = rt_t == 0
            last = rt_t == n_rt - 1
            r0_t = rt_t * th

            @pl.when(first)
            def _():
                go(pltpu.make_async_copy(
                    x_hbm.at[n, pl.ds(0, th + 1)], dst(s, 1, th + 1),
                    sem.at[s]))

            @pl.when(jnp.logical_and(jnp.logical_not(first),
                                     jnp.logical_not(last)))
            def _():
                go(pltpu.make_async_copy(
                    x_hbm.at[n, pl.ds(r0_t - 1, th + 2)], dst(s, 0, th + 2),
                    sem.at[s]))

            @pl.when(jnp.logical_and(last, jnp.logical_not(first)))
            def _():
                go(pltpu.make_async_copy(
                    x_hbm.at[n, pl.ds(r0_t - 1, th + 1)], dst(s, 0, th + 1),
                    sem.at[s]))

        @pl.when(rt == 0)
        def _():
            halo(0, 0, 'start')

        @pl.when(rt + 1 < n_rt)
        def _():
            halo(rt + 1, 1 - slot, 'start')

        halo(rt, slot, 'wait')

        @pl.when(rt == 0)
        def _():
            x_vmem[0, 0:1] = jnp.zeros((1, wp, cin), dt)

        @pl.when(rt == n_rt - 1)
        def _():
            x_vmem[(n_rt - 1) % 2, th + 1:th + 2] = jnp.zeros(
                (1, wp, cin), dt)

        if pad_scr:
            x_vmem[slot, :, col0 - 1:col0] = jnp.zeros((th + 2, 1, cin), dt)
            x_vmem[slot, :, col0 + w_out:col0 + w_out + 1] = jnp.zeros(
                (th + 2, 1, cin), dt)

        def rows(a, m):
            return x_vmem[slot, a:a + m]

    bias = b_ref[...]
    tco = b_ref.shape[-1]
    if fold:
        w_all = w_ref[...]
    elif dy_only:
        w_taps = [w_ref[t] for t in range(3)]
    else:
        w_taps = [w_ref[t] for t in range(9)]

    zcol = jnp.zeros((1, cin), dt)

    def prow(rr):
        return jnp.concatenate([zcol, rows(rr, 1).reshape(wp, cin), zcol],
                               axis=0)

    for g in range(th // rg):
        r0 = g * rg
        if dy_only:
            acc = jnp.zeros((rg * w_out, tco), jnp.float32)
            for dy in range(3):
                l = rows(r0 + dy, rg).reshape(rg * w_out, cin)
                acc = acc + jnp.dot(l, w_taps[dy],
                                    preferred_element_type=jnp.float32)
        elif pad_scr:
            acc = jnp.zeros((rg * w_out, tco), jnp.float32)
            for t, (dy, dx) in enumerate([(a, b) for a in range(3)
                                          for b in range(3)]):
                lhs = rows(r0 + dy, rg)[:, col0 - 1 + dx:col0 - 1 + dx + w_out]
                acc = acc + jnp.dot(lhs.reshape(rg * w_out, cin), w_taps[t],
                                    preferred_element_type=jnp.float32)
        elif fold:
            pr = [prow(r0 + i) for i in range(rg + 2)]
            lhs = jnp.concatenate(
                [jnp.concatenate([pr[i + dy][dx:dx + w_out]
                                  for dy in range(3) for dx in range(3)],
                                 axis=-1)
                 for i in range(rg)], axis=0)
            acc = jnp.dot(lhs, w_all, preferred_element_type=jnp.float32)
        else:
            pr = [prow(r0 + i) for i in range(rg + 2)]
            acc = jnp.zeros((rg * w_out, tco), jnp.float32)
            t = 0
            for dy in range(3):
                for dx in range(3):
                    if rg == 1:
                        l = pr[dy][dx:dx + w_out]
                    else:
                        l = jnp.concatenate(
                            [pr[i + dy][dx:dx + w_out] for i in range(rg)],
                            axis=0)
                    acc = acc + jnp.dot(l, w_taps[t],
                                        preferred_element_type=jnp.float32)
                    t += 1
        acc = acc + bias
        if emit_preact:
            f_ref[pl.ds(r0, rg)] = acc.reshape(rg, w_out, tco)
        y = jnp.maximum(acc, 0.0)
        if do_pool:
            y4 = y.reshape(rg // 2, 2, w_out, tco)
            m = jnp.maximum(y4[:, 0], y4[:, 1])
            m4 = m.reshape(rg // 2, w_out // 2, 2, tco)
            m = jnp.maximum(m4[:, :, 0, :], m4[:, :, 1, :])
            y_ref[pl.ds(r0 // 2, rg // 2)] = m.astype(y_ref.dtype)
        else:
            y_ref[pl.ds(r0, rg)] = y.reshape(rg, w_out, tco).astype(y_ref.dtype)


def _conv(x, w, b, *, preact, pool, dy_only=False):
    n, h, wd, cin = x.shape
    cout = w.shape[-1]

    th = h
    for t in (56, 32, 28, 16, 8):
        if h % t == 0:
            th = t
            break
    n_rt = h // th
    tco = min(cout, 128)
    n_co = cout // tco
    pad_scr = False
    fold = (cin % 128 == 0) and not pad_scr and not dy_only

    rg = th
    for d in range(1, th + 1):
        if th % d == 0 and d * wd >= 112:
            rg = d
            break
    if pool and rg % 2:
        rg *= 2
    assert th % rg == 0 and (not pool or rg % 2 == 0)

    wb = w.astype(jnp.bfloat16)
    if fold:
        w_in = wb.reshape(9 * cin, cout)
        w_spec = pl.BlockSpec((9 * cin, tco), lambda i, j, k: (0, j))
    elif dy_only:
        w_in = wb.reshape(3, cin, cout)
        w_spec = pl.BlockSpec((3, cin, tco), lambda i, j, k: (0, 0, j))
    else:
        w_in = wb.reshape(9, cin, cout)
        w_spec = pl.BlockSpec((9, cin, tco), lambda i, j, k: (0, 0, j))
    b_in = b.reshape(1, cout)

    ho, wo = (h // 2, wd // 2) if pool else (h, wd)
    tho = th // 2 if pool else th
    y_sds = jax.ShapeDtypeStruct((n, ho, wo, cout), jnp.bfloat16)
    y_spec = pl.BlockSpec((None, tho, wo, tco), lambda i, j, k: (i, k, 0, j))
    if preact:
        f_sds = jax.ShapeDtypeStruct((n, h, wd, cout), jnp.float32)
        f_spec = pl.BlockSpec((None, th, wd, tco), lambda i, j, k: (i, k, 0, j))
        out_shape = (y_sds, f_sds)
        out_specs = (y_spec, f_spec)
    else:
        out_shape = y_sds
        out_specs = y_spec

    wp = wd + 16 if pad_scr else wd
    body = functools.partial(_conv_body, th=th, n_rt=n_rt, w_out=wd, cin=cin,
                             rg=rg, fold=fold, pad_scr=pad_scr,
                             dy_only=dy_only, emit_preact=preact, do_pool=pool)
    outs = pl.pallas_call(
        body,
        out_shape=out_shape,
        grid_spec=pltpu.PrefetchScalarGridSpec(
            num_scalar_prefetch=0,
            grid=(n, n_co, n_rt),
            in_specs=[
                pl.BlockSpec(memory_space=pl.ANY),
                w_spec,
                pl.BlockSpec((1, tco), lambda i, j, k: (0, j)),
            ],
            out_specs=out_specs,
            scratch_shapes=[
                pltpu.VMEM((th + 2, wp, cin), jnp.bfloat16) if n_rt == 1
                else pltpu.VMEM((2, th + 2, wp, cin), jnp.bfloat16),
                pltpu.SemaphoreType.DMA if n_rt == 1
                else pltpu.SemaphoreType.DMA((2,)),
            ]),
        compiler_params=pltpu.CompilerParams(
            dimension_semantics=("parallel", "parallel", "arbitrary")),
    )(x, w_in, b_in)
    if preact:
        return outs[0], outs[1]
    return outs, None


_PLAN = [(True, False), (False, True),
         (True, False), (False, True),
         (True, False), (False, False), (False, False), (False, True),
         (True, False), (False, False), (False, False), (False, True),
         (True, False)]


def kernel(x, w0, b0, w1, b1, w2, b2, w3, b3, w4, b4, w5, b5, w6, b6, w7, b7,
           w8, b8, w9, b9, w10, b10, w11, b11, w12, b12, w13, b13, w14, b14,
           w15, b15):
    ws = [w0, w1, w2, w3, w4, w5, w6, w7, w8, w9, w10, w11, w12]
    bs = [b0, b1, b2, b3, b4, b5, b6, b7, b8, b9, b10, b11, b12]
    x = jnp.transpose(x, (0, 2, 3, 1)).astype(jnp.bfloat16)
    wd = x.shape[2]
    xp = jnp.pad(x, ((0, 0), (0, 0), (1, 1), (0, 0)))
    x = jnp.concatenate([xp[:, :, 0:wd], xp[:, :, 1:wd + 1],
                         xp[:, :, 2:wd + 2]], axis=-1)
    feats = []
    for li, (pre, po) in enumerate(_PLAN):
        x, f = _conv(x, ws[li], bs[li], preact=pre, pool=po,
                     dy_only=(li == 0))
        if pre:
            feats.append(jnp.transpose(f, (0, 3, 1, 2)))
    return tuple(feats)
